# Initial kernel scaffold; baseline (speedup 1.0000x reference)
import jax
import jax.numpy as jnp
from jax.experimental import pallas as pl


def kernel(x, edge_index, edge_attr, W, attn, edge_W, edge_b):
    raise NotImplementedError("write your pallas kernel here")


# speedup vs baseline: 21.7061x; 21.7061x over previous
import functools

import jax
import jax.numpy as jnp
from jax import lax
from jax.experimental import pallas as pl
from jax.experimental.pallas import tpu as pltpu
from jax.experimental.pallas import tpu_sc as plsc

N, E, D_IN, D_OUT, D_EDGE = 10000, 320000, 128, 128, 16
NC, NS, L = 2, 16, 16
NW = NC * NS
CH = E // NW
G = CH // L
NPAD = 10240
STRIPE = NPAD // NS


def _dense_body(x_ref, w_ref, a2_ref, wh_ref, s_ref):
    wh = lax.dot_general(x_ref[...], w_ref[...],
                         (((1,), (1,)), ((), ())),
                         preferred_element_type=jnp.float32)
    wh_ref[...] = wh
    s_ref[...] = lax.dot_general(a2_ref[...], wh,
                                 (((1,), (1,)), ((), ())),
                                 preferred_element_type=jnp.float32)


def _edge_term_body(ea_ref, ew_ref, a3_ref, eb_ref, t_ref):
    a3 = a3_ref[...]
    v = lax.dot_general(a3, ew_ref[...],
                        (((1,), (0,)), ((), ())),
                        preferred_element_type=jnp.float32)
    c = jnp.sum(eb_ref[...] * a3[0, :])
    t_ref[...] = jnp.sum(ea_ref[...] * v, axis=1, keepdims=True) + c


def _final_body(wh_ref, s_ref, o_ref):
    s = s_ref[...]
    tot = jnp.sum(s, axis=1)
    a = tot / jnp.maximum(tot, 1e-10)
    o_ref[...] = wh_ref[...] * a[:, None]


def _sc_body(src_hbm, dst_hbm, t_hbm, s1_hbm, s2_hbm, s_out_hbm,
             src_v, dst_v, t_v, p_v, s1_v, s2_v, z_v, s_sh):
    cid = lax.axis_index("c")
    sid = lax.axis_index("s")
    wid = sid * NC + cid
    base = wid * CH
    pltpu.sync_copy(src_hbm.at[pl.ds(base, CH)], src_v)
    pltpu.sync_copy(dst_hbm.at[pl.ds(base, CH)], dst_v)
    pltpu.sync_copy(t_hbm.at[pl.ds(base, CH)], t_v)
    pltpu.sync_copy(s1_hbm, s1_v)
    pltpu.sync_copy(s2_hbm, s2_v)

    def zbody(i, carry):
        z_v[pl.ds(i * L, L)] = jnp.zeros((L,), jnp.float32)
        return carry
    lax.fori_loop(0, STRIPE // L, zbody, 0)
    pltpu.sync_copy(z_v, s_sh.at[pl.ds(sid * STRIPE, STRIPE)])

    def body(i, carry):
        sl = pl.ds(i * L, L)
        g1 = plsc.load_gather(s1_v, [src_v[sl]])
        g2 = plsc.load_gather(s2_v, [dst_v[sl]])
        e = g1 + g2 + t_v[sl]
        e = jnp.maximum(e, 0.2 * e)
        e = jnp.minimum(e, 30.0)
        p_v[sl] = jnp.exp(e)
        return carry
    lax.fori_loop(0, G, body, 0)

    plsc.subcore_barrier()
    pltpu.sync_copy(p_v, s_sh.at[dst_v], add=True)
    plsc.subcore_barrier()
    pltpu.sync_copy(s_sh.at[pl.ds(sid * STRIPE, STRIPE)],
                    s_out_hbm.at[cid, pl.ds(sid * STRIPE, STRIPE)])


@jax.jit
def kernel(x, edge_index, edge_attr, W, attn, edge_W, edge_b):
    a2 = attn[:, :2 * D_OUT].reshape(2, D_OUT)
    a3 = attn[:, 2 * D_OUT:]

    wh, s12 = pl.pallas_call(
        _dense_body,
        out_shape=(jax.ShapeDtypeStruct((N, D_OUT), jnp.float32),
                   jax.ShapeDtypeStruct((2, N), jnp.float32)),
    )(x, W, a2)

    EB = 8000
    t = pl.pallas_call(
        _edge_term_body,
        grid=(E // EB,),
        in_specs=[pl.BlockSpec((EB, D_EDGE), lambda i: (i, 0)),
                  pl.BlockSpec((D_EDGE, D_EDGE), lambda i: (0, 0)),
                  pl.BlockSpec((1, D_EDGE), lambda i: (0, 0)),
                  pl.BlockSpec((D_EDGE,), lambda i: (0,))],
        out_specs=pl.BlockSpec((EB, 1), lambda i: (i, 0)),
        out_shape=jax.ShapeDtypeStruct((E, 1), jnp.float32),
    )(edge_attr, edge_W, a3, edge_b)
    t_flat = t.reshape(E)

    sc = pl.kernel(
        _sc_body,
        out_type=jax.ShapeDtypeStruct((NC, NPAD), jnp.float32),
        mesh=plsc.VectorSubcoreMesh(core_axis_name="c", subcore_axis_name="s"),
        compiler_params=pltpu.CompilerParams(needs_layout_passes=False),
        scratch_types=[
            pltpu.VMEM((CH,), jnp.int32),
            pltpu.VMEM((CH,), jnp.int32),
            pltpu.VMEM((CH,), jnp.float32),
            pltpu.VMEM((CH,), jnp.float32),
            pltpu.VMEM((N,), jnp.float32),
            pltpu.VMEM((N,), jnp.float32),
            pltpu.VMEM((STRIPE,), jnp.float32),
            pltpu.VMEM_SHARED((NPAD,), jnp.float32),
        ],
    )
    s_pair = sc(edge_index[0], edge_index[1], t_flat, s12[0], s12[1])

    s_t = jnp.transpose(s_pair)[:N]

    NB = 2000
    out = pl.pallas_call(
        _final_body,
        grid=(N // NB,),
        in_specs=[pl.BlockSpec((NB, D_OUT), lambda i: (i, 0)),
                  pl.BlockSpec((NB, 2), lambda i: (i, 0))],
        out_specs=pl.BlockSpec((NB, D_OUT), lambda i: (i, 0)),
        out_shape=jax.ShapeDtypeStruct((N, D_OUT), jnp.float32),
    )(wh, s_t)
    return out
